# baseline (speedup 1.0000x reference)
import jax
import jax.numpy as jnp
from jax import lax
from jax.experimental import pallas as pl
from jax.experimental.pallas import tpu as pltpu
from jax.experimental.pallas import tpu_sc as plsc

N_GRAPH = 256
N_NODE = 128
N_FEAT = 9
EMB = 128
OUT_ROWS = N_GRAPH * (N_NODE + 1)

NC = 2
NS = 16
NW = NC * NS

GPW = N_GRAPH // NW
CHUNK = N_NODE
IDXC = N_FEAT * CHUNK
NBUF = 3


def _sc_body(x_hbm, deg_hbm, node_hbm, degt_hbm, tok_hbm, out_hbm,
             nix, dgx, rix, posb, acc_v, raw_sh, degt_sh, tok_rows_v,
             tok_idx_v, semi, semp, semd, semg, semo):
    cid = lax.axis_index("c")
    sid = lax.axis_index("s")
    wid = sid * NC + cid
    lane = lax.iota(jnp.int32, 16)

    @pl.when(sid == 0)
    def _():
        pltpu.sync_copy(degt_hbm, degt_sh)
    plsc.subcore_barrier()

    for b in range(NBUF):
        base = (sid * NBUF + b) * IDXC
        for j in range(N_FEAT):
            for v in range(CHUNK // 16):
                posb[b][pl.ds(j * CHUNK + v * 16, 16)] = (
                    base + (v * 16 + lane) * N_FEAT + j)

    def fetch_idx(c):
        b = c % NBUF
        g0 = wid * GPW + c
        slot = (sid * NBUF + b) * IDXC
        pltpu.async_copy(x_hbm.at[pl.ds(g0 * IDXC, IDXC)],
                         raw_sh.at[pl.ds(slot, IDXC)], semi[b])
        pltpu.async_copy(deg_hbm.at[pl.ds(g0 * CHUNK, CHUNK)], dgx[b],
                         semi[b])

    def drain_idx(c):
        b = c % NBUF
        slot = (sid * NBUF + b) * IDXC
        pltpu.make_async_copy(x_hbm.at[pl.ds(0, IDXC)],
                              raw_sh.at[pl.ds(slot, IDXC)], semi[b]).wait()
        pltpu.make_async_copy(deg_hbm.at[pl.ds(0, CHUNK)], dgx[b],
                              semi[b]).wait()

    def issue_pos(c):
        b = c % NBUF
        for t in range(N_FEAT):
            pltpu.async_copy(
                raw_sh.at[posb[b].at[pl.ds(t * CHUNK, CHUNK)]],
                nix[b].at[pl.ds(t * CHUNK, CHUNK)], semp[b])

    def drain_pos(c):
        b = c % NBUF
        for t in range(N_FEAT):
            pltpu.make_async_copy(
                raw_sh.at[posb[b].at[pl.ds(t * CHUNK, CHUNK)]],
                nix[b].at[pl.ds(t * CHUNK, CHUNK)], semp[b]).wait()

    def issue_deg(c):
        b = c % NBUF
        pltpu.async_copy(degt_sh.at[dgx[b]], acc_v.at[b], semd[b])

    def drain_deg(c):
        b = c % NBUF
        pltpu.make_async_copy(degt_sh.at[dgx[b]], acc_v.at[b],
                              semd[b]).wait()

    def issue_nodes(c):
        b = c % NBUF
        for j in range(N_FEAT):
            pltpu.async_copy(
                node_hbm.at[nix[b].at[pl.ds(j * CHUNK, CHUNK)]],
                acc_v.at[b], semg[b], add=True)

    def drain_nodes(c):
        b = c % NBUF
        for j in range(N_FEAT):
            pltpu.make_async_copy(
                node_hbm.at[nix[b].at[pl.ds(j * CHUNK, CHUNK)]],
                acc_v.at[b], semg[b]).wait()

    def issue_scatter(c):
        b = c % NBUF
        row0 = (wid * GPW + c) * (N_NODE + 1) + 1
        for v in range(CHUNK // 16):
            rix[b][pl.ds(v * 16, 16)] = row0 + v * 16 + lane
        pltpu.async_copy(acc_v.at[b], out_hbm.at[rix[b]], semo[b])

    def drain_scatter(c):
        b = c % NBUF
        pltpu.make_async_copy(acc_v.at[b], out_hbm.at[rix[b]],
                              semo[b]).wait()

    fetch_idx(0)
    fetch_idx(1)
    drain_idx(0)
    issue_pos(0)
    issue_deg(0)
    drain_pos(0)
    drain_deg(0)
    issue_nodes(0)

    pltpu.sync_copy(tok_hbm, tok_rows_v.at[pl.ds(0, 1)])
    for v in range(EMB // 16):
        sl = pl.ds(v * 16, 16)
        tv = tok_rows_v[0, sl]
        for i in range(1, 16):
            tok_rows_v[i, sl] = tv
    tok_idx_v[pl.ds(0, 16)] = (wid * GPW + lane % GPW) * (N_NODE + 1)
    pltpu.async_copy(tok_rows_v, out_hbm.at[tok_idx_v], semd[1]).wait()

    for c in range(GPW):
        if c >= 1:
            drain_scatter(c - 1)
        if c + 2 < GPW:
            fetch_idx(c + 2)
        if c + 1 < GPW:
            drain_idx(c + 1)
            issue_pos(c + 1)
            issue_deg(c + 1)
        drain_nodes(c)
        issue_scatter(c)
        if c + 1 < GPW:
            drain_pos(c + 1)
            drain_deg(c + 1)
            issue_nodes(c + 1)
    drain_scatter(GPW - 1)


@jax.jit
def _graph_node_features(x_flat, deg_flat, node_table, degree_table,
                         graph_token):
    mesh = plsc.VectorSubcoreMesh(core_axis_name="c", subcore_axis_name="s")
    out = pl.kernel(
        _sc_body,
        out_type=jax.ShapeDtypeStruct((OUT_ROWS, EMB), jnp.float32),
        mesh=mesh,
        scratch_types=[
            [pltpu.VMEM((IDXC,), jnp.int32) for _ in range(NBUF)],
            [pltpu.VMEM((CHUNK,), jnp.int32) for _ in range(NBUF)],
            [pltpu.VMEM((CHUNK,), jnp.int32) for _ in range(NBUF)],
            [pltpu.VMEM((IDXC,), jnp.int32) for _ in range(NBUF)],
            pltpu.VMEM((NBUF, CHUNK, EMB), jnp.float32),
            pltpu.VMEM_SHARED((NS * NBUF * IDXC,), jnp.int32),
            pltpu.VMEM_SHARED((512, EMB), jnp.float32),
            pltpu.VMEM((16, EMB), jnp.float32),
            pltpu.VMEM((16,), jnp.int32),
            [pltpu.SemaphoreType.DMA for _ in range(NBUF)],
            [pltpu.SemaphoreType.DMA for _ in range(NBUF)],
            [pltpu.SemaphoreType.DMA for _ in range(NBUF)],
            [pltpu.SemaphoreType.DMA for _ in range(NBUF)],
            [pltpu.SemaphoreType.DMA for _ in range(NBUF)],
        ],
    )(x_flat, deg_flat, node_table, degree_table, graph_token)
    return out.reshape(N_GRAPH, N_NODE + 1, EMB)


def kernel(x, degree, node_table, degree_table, graph_token):
    x_flat = x.reshape(-1).astype(jnp.int32)
    deg_flat = degree.reshape(-1).astype(jnp.int32)
    return _graph_node_features(x_flat, deg_flat, node_table, degree_table,
                                graph_token)


# speedup vs baseline: 1.3754x; 1.3754x over previous
import jax
import jax.numpy as jnp
from jax import lax
from jax.experimental import pallas as pl
from jax.experimental.pallas import tpu as pltpu
from jax.experimental.pallas import tpu_sc as plsc

N_GRAPH = 256
N_NODE = 128
N_FEAT = 9
EMB = 128
OUT_ROWS = N_GRAPH * (N_NODE + 1)
PITCH = 136

NC = 2
NS = 16
NW = NC * NS

GPW = N_GRAPH // NW
CHUNK = N_NODE
IDXC = N_FEAT * CHUNK
NBUF = 3


def _sc_body(xt_hbm, deg_hbm, node_hbm, degt_hbm, tok_hbm, out_hbm,
             nix, dgx, rix, acc_v, degt_sh, tok_rows_v, tok_idx_v,
             semi, semd, semg, semo):
    cid = lax.axis_index("c")
    sid = lax.axis_index("s")
    wid = sid * NC + cid
    lane = lax.iota(jnp.int32, 16)

    @pl.when(sid == 0)
    def _():
        pltpu.sync_copy(degt_hbm, degt_sh)
    plsc.subcore_barrier()

    def fetch_idx(c):
        b = c % NBUF
        g0 = wid * GPW + c
        pltpu.async_copy(xt_hbm.at[pl.ds(g0 * IDXC, IDXC)], nix[b], semi[b])
        pltpu.async_copy(deg_hbm.at[pl.ds(g0 * CHUNK, CHUNK)], dgx[b],
                         semi[b])

    def drain_idx(c):
        b = c % NBUF
        pltpu.make_async_copy(xt_hbm.at[pl.ds(0, IDXC)], nix[b],
                              semi[b]).wait()
        pltpu.make_async_copy(deg_hbm.at[pl.ds(0, CHUNK)], dgx[b],
                              semi[b]).wait()

    def issue_deg(c):
        b = c % NBUF
        pltpu.async_copy(degt_sh.at[dgx[b]], acc_v.at[b], semd[b])

    def drain_deg(c):
        b = c % NBUF
        pltpu.make_async_copy(degt_sh.at[dgx[b]], acc_v.at[b],
                              semd[b]).wait()

    def issue_nodes(c):
        b = c % NBUF
        for j in range(N_FEAT):
            pltpu.async_copy(
                node_hbm.at[nix[b].at[pl.ds(j * CHUNK, CHUNK)]],
                acc_v.at[b], semg[b], add=True)

    def drain_nodes(c):
        b = c % NBUF
        for j in range(N_FEAT):
            pltpu.make_async_copy(
                node_hbm.at[nix[b].at[pl.ds(j * CHUNK, CHUNK)]],
                acc_v.at[b], semg[b]).wait()

    def issue_scatter(c):
        b = c % NBUF
        row0 = (wid * GPW + c) * PITCH + 1
        for v in range(CHUNK // 16):
            rix[b][pl.ds(v * 16, 16)] = row0 + v * 16 + lane
        pltpu.async_copy(acc_v.at[b], out_hbm.at[rix[b]], semo[b])

    def drain_scatter(c):
        b = c % NBUF
        pltpu.make_async_copy(acc_v.at[b], out_hbm.at[rix[b]],
                              semo[b]).wait()

    fetch_idx(0)
    fetch_idx(1)
    drain_idx(0)
    issue_deg(0)
    drain_deg(0)
    issue_nodes(0)

    pltpu.sync_copy(tok_hbm, tok_rows_v.at[pl.ds(0, 1)])
    for v in range(EMB // 16):
        sl = pl.ds(v * 16, 16)
        tv = tok_rows_v[0, sl]
        for i in range(1, 16):
            tok_rows_v[i, sl] = tv
    tok_idx_v[pl.ds(0, 16)] = (wid * GPW + lane % GPW) * PITCH
    pltpu.async_copy(tok_rows_v, out_hbm.at[tok_idx_v], semd[1]).wait()

    for c in range(GPW):
        if c >= 1:
            drain_scatter(c - 1)
        if c + 2 < GPW:
            fetch_idx(c + 2)
        if c + 1 < GPW:
            drain_idx(c + 1)
            issue_deg(c + 1)
        drain_nodes(c)
        issue_scatter(c)
        if c + 1 < GPW:
            drain_deg(c + 1)
            issue_nodes(c + 1)
    drain_scatter(GPW - 1)


@jax.jit
def _graph_node_features(xt_flat, deg_flat, node_table, degree_table,
                         graph_token):
    mesh = plsc.VectorSubcoreMesh(core_axis_name="c", subcore_axis_name="s")
    out = pl.kernel(
        _sc_body,
        out_type=jax.ShapeDtypeStruct((N_GRAPH * PITCH, EMB), jnp.float32),
        mesh=mesh,
        scratch_types=[
            [pltpu.VMEM((IDXC,), jnp.int32) for _ in range(NBUF)],
            [pltpu.VMEM((CHUNK,), jnp.int32) for _ in range(NBUF)],
            [pltpu.VMEM((CHUNK,), jnp.int32) for _ in range(NBUF)],
            pltpu.VMEM((NBUF, CHUNK, EMB), jnp.float32),
            pltpu.VMEM_SHARED((512, EMB), jnp.float32),
            pltpu.VMEM((16, EMB), jnp.float32),
            pltpu.VMEM((16,), jnp.int32),
            [pltpu.SemaphoreType.DMA for _ in range(NBUF)],
            [pltpu.SemaphoreType.DMA for _ in range(NBUF)],
            [pltpu.SemaphoreType.DMA for _ in range(NBUF)],
            [pltpu.SemaphoreType.DMA for _ in range(NBUF)],
        ],
    )(xt_flat, deg_flat, node_table, degree_table, graph_token)
    return out.reshape(N_GRAPH, PITCH, EMB)[:, :N_NODE + 1, :]


def kernel(x, degree, node_table, degree_table, graph_token):
    xt_flat = x.astype(jnp.int32).transpose(0, 2, 1).reshape(-1)
    deg_flat = degree.reshape(-1).astype(jnp.int32)
    return _graph_node_features(xt_flat, deg_flat, node_table, degree_table,
                                graph_token)


# speedup vs baseline: 1.3964x; 1.0152x over previous
import jax
import jax.numpy as jnp
from jax import lax
from jax.experimental import pallas as pl
from jax.experimental.pallas import tpu as pltpu
from jax.experimental.pallas import tpu_sc as plsc

N_GRAPH = 256
N_NODE = 128
N_FEAT = 9
EMB = 128
OUT_ROWS = N_GRAPH * (N_NODE + 1)
PITCH = 136

NC = 2
NS = 16
NW = NC * NS

GPW = N_GRAPH // NW
CHUNK = N_NODE
IDXC = N_FEAT * CHUNK
NBUF = 3


def _sc_body(xt_hbm, deg_hbm, node_hbm, degt_hbm, tok_hbm, out3_hbm,
             nix, dgx, rix, acc_v, degt_sh, tok_rows_v, tok_idx_v,
             semi, semd, semg, semo):
    cid = lax.axis_index("c")
    sid = lax.axis_index("s")
    wid = sid * NC + cid
    lane = lax.iota(jnp.int32, 16)

    @pl.when(sid == 0)
    def _():
        pltpu.sync_copy(degt_hbm, degt_sh)
    plsc.subcore_barrier()

    def fetch_idx(c):
        b = c % NBUF
        g0 = wid * GPW + c
        pltpu.async_copy(xt_hbm.at[pl.ds(g0 * IDXC, IDXC)], nix[b], semi[b])
        pltpu.async_copy(deg_hbm.at[pl.ds(g0 * CHUNK, CHUNK)], dgx[b],
                         semi[b])

    def drain_idx(c):
        b = c % NBUF
        pltpu.make_async_copy(xt_hbm.at[pl.ds(0, IDXC)], nix[b],
                              semi[b]).wait()
        pltpu.make_async_copy(deg_hbm.at[pl.ds(0, CHUNK)], dgx[b],
                              semi[b]).wait()

    def issue_deg(c):
        b = c % NBUF
        pltpu.async_copy(degt_sh.at[dgx[b]], acc_v.at[b], semd[b])

    def drain_deg(c):
        b = c % NBUF
        pltpu.make_async_copy(degt_sh.at[dgx[b]], acc_v.at[b],
                              semd[b]).wait()

    def issue_nodes(c):
        b = c % NBUF
        for j in range(N_FEAT):
            pltpu.async_copy(
                node_hbm.at[nix[b].at[pl.ds(j * CHUNK, CHUNK)]],
                acc_v.at[b], semg[b], add=True)

    def drain_nodes(c):
        b = c % NBUF
        for j in range(N_FEAT):
            pltpu.make_async_copy(
                node_hbm.at[nix[b].at[pl.ds(j * CHUNK, CHUNK)]],
                acc_v.at[b], semg[b]).wait()

    for v in range(CHUNK // 16):
        rix[0][pl.ds(v * 16, 16)] = 1 + v * 16 + lane

    def issue_scatter(c):
        b = c % NBUF
        g = wid * GPW + c
        pltpu.async_copy(acc_v.at[b], out3_hbm.at[g].at[rix[0]], semo[b])

    def drain_scatter(c):
        b = c % NBUF
        g = wid * GPW + c
        pltpu.make_async_copy(acc_v.at[b], out3_hbm.at[g].at[rix[0]],
                              semo[b]).wait()

    fetch_idx(0)
    fetch_idx(1)
    drain_idx(0)
    issue_deg(0)
    drain_deg(0)
    issue_nodes(0)

    pltpu.sync_copy(tok_hbm, tok_rows_v.at[pl.ds(0, 1)])
    for v in range(EMB // 16):
        sl = pl.ds(v * 16, 16)
        tv = tok_rows_v[0, sl]
        for i in range(1, 16):
            tok_rows_v[i, sl] = tv
    tok_idx_v[pl.ds(0, 16)] = lane - lane
    for gi in range(GPW):
        g = wid * GPW + gi
        pltpu.async_copy(tok_rows_v, out3_hbm.at[g].at[tok_idx_v], semd[1])
    for gi in range(GPW):
        g = wid * GPW + gi
        pltpu.make_async_copy(tok_rows_v, out3_hbm.at[g].at[tok_idx_v],
                              semd[1]).wait()

    for c in range(GPW):
        if c >= 1:
            drain_scatter(c - 1)
        if c + 2 < GPW:
            fetch_idx(c + 2)
        if c + 1 < GPW:
            drain_idx(c + 1)
            issue_deg(c + 1)
        drain_nodes(c)
        issue_scatter(c)
        if c + 1 < GPW:
            drain_deg(c + 1)
            issue_nodes(c + 1)
    drain_scatter(GPW - 1)


@jax.jit
def _graph_node_features(xt_flat, deg_flat, node_table, degree_table,
                         graph_token):
    mesh = plsc.VectorSubcoreMesh(core_axis_name="c", subcore_axis_name="s")
    out = pl.kernel(
        _sc_body,
        out_type=jax.ShapeDtypeStruct((N_GRAPH, N_NODE + 1, EMB),
                                       jnp.float32),
        mesh=mesh,
        scratch_types=[
            [pltpu.VMEM((IDXC,), jnp.int32) for _ in range(NBUF)],
            [pltpu.VMEM((CHUNK,), jnp.int32) for _ in range(NBUF)],
            [pltpu.VMEM((CHUNK,), jnp.int32) for _ in range(1)],
            pltpu.VMEM((NBUF, CHUNK, EMB), jnp.float32),
            pltpu.VMEM_SHARED((512, EMB), jnp.float32),
            pltpu.VMEM((16, EMB), jnp.float32),
            pltpu.VMEM((16,), jnp.int32),
            [pltpu.SemaphoreType.DMA for _ in range(NBUF)],
            [pltpu.SemaphoreType.DMA for _ in range(NBUF)],
            [pltpu.SemaphoreType.DMA for _ in range(NBUF)],
            [pltpu.SemaphoreType.DMA for _ in range(NBUF)],
        ],
    )(xt_flat, deg_flat, node_table, degree_table, graph_token)
    return out


def kernel(x, degree, node_table, degree_table, graph_token):
    xt_flat = x.astype(jnp.int32).transpose(0, 2, 1).reshape(-1)
    deg_flat = degree.reshape(-1).astype(jnp.int32)
    return _graph_node_features(xt_flat, deg_flat, node_table, degree_table,
                                graph_token)


# speedup vs baseline: 1.4035x; 1.0051x over previous
import jax
import jax.numpy as jnp
from jax import lax
from jax.experimental import pallas as pl
from jax.experimental.pallas import tpu as pltpu
from jax.experimental.pallas import tpu_sc as plsc

N_GRAPH = 256
N_NODE = 128
N_FEAT = 9
EMB = 128
NC = 2
NS = 16
NW = NC * NS

GPW = N_GRAPH // NW
CHUNK = N_NODE
IDXC = N_FEAT * CHUNK
NBUF = 3


def _sc_body(xt_hbm, deg_hbm, node_hbm, degt_hbm, tok_hbm, out3_hbm,
             nix, dgx, rix, acc_v, degt_sh, tok_rows_v, tok_idx_v,
             semi, semd, semg, semo):
    cid = lax.axis_index("c")
    sid = lax.axis_index("s")
    wid = sid * NC + cid
    lane = lax.iota(jnp.int32, 16)

    @pl.when(sid == 0)
    def _():
        pltpu.sync_copy(degt_hbm, degt_sh)
    plsc.subcore_barrier()

    def fetch_idx(c):
        b = c % NBUF
        g0 = wid * GPW + c
        pltpu.async_copy(xt_hbm.at[pl.ds(g0 * IDXC, IDXC)], nix[b], semi[b])
        pltpu.async_copy(deg_hbm.at[pl.ds(g0 * CHUNK, CHUNK)], dgx[b],
                         semi[b])

    def drain_idx(c):
        b = c % NBUF
        pltpu.make_async_copy(xt_hbm.at[pl.ds(0, IDXC)], nix[b],
                              semi[b]).wait()
        pltpu.make_async_copy(deg_hbm.at[pl.ds(0, CHUNK)], dgx[b],
                              semi[b]).wait()

    def issue_deg(c):
        b = c % NBUF
        pltpu.async_copy(degt_sh.at[dgx[b]], acc_v.at[b], semd[b])

    def drain_deg(c):
        b = c % NBUF
        pltpu.make_async_copy(degt_sh.at[dgx[b]], acc_v.at[b],
                              semd[b]).wait()

    def issue_nodes(c):
        b = c % NBUF
        for j in range(N_FEAT):
            pltpu.async_copy(
                node_hbm.at[nix[b].at[pl.ds(j * CHUNK, CHUNK)]],
                acc_v.at[b], semg[b], add=True)

    def drain_nodes(c):
        b = c % NBUF
        for j in range(N_FEAT):
            pltpu.make_async_copy(
                node_hbm.at[nix[b].at[pl.ds(j * CHUNK, CHUNK)]],
                acc_v.at[b], semg[b]).wait()

    for v in range(CHUNK // 16):
        rix[0][pl.ds(v * 16, 16)] = 1 + v * 16 + lane

    def issue_scatter(c):
        b = c % NBUF
        g = wid * GPW + c
        pltpu.async_copy(acc_v.at[b], out3_hbm.at[g].at[rix[0]], semo[b])

    def drain_scatter(c):
        b = c % NBUF
        g = wid * GPW + c
        pltpu.make_async_copy(acc_v.at[b], out3_hbm.at[g].at[rix[0]],
                              semo[b]).wait()

    fetch_idx(0)
    fetch_idx(1)
    drain_idx(0)
    issue_deg(0)
    drain_deg(0)
    issue_nodes(0)

    pltpu.sync_copy(tok_hbm, tok_rows_v.at[pl.ds(0, 1)])
    for v in range(EMB // 16):
        sl = pl.ds(v * 16, 16)
        tv = tok_rows_v[0, sl]
        for i in range(1, 16):
            tok_rows_v[i, sl] = tv
    tok_idx_v[pl.ds(0, 16)] = lane - lane
    for gi in range(GPW):
        g = wid * GPW + gi
        pltpu.async_copy(tok_rows_v, out3_hbm.at[g].at[tok_idx_v], semd[1])
    for gi in range(GPW):
        g = wid * GPW + gi
        pltpu.make_async_copy(tok_rows_v, out3_hbm.at[g].at[tok_idx_v],
                              semd[1]).wait()

    for c in range(GPW):
        if c >= 1:
            drain_scatter(c - 1)
        if c + 2 < GPW:
            fetch_idx(c + 2)
        if c + 1 < GPW:
            drain_idx(c + 1)
            issue_deg(c + 1)
        drain_nodes(c)
        issue_scatter(c)
        if c + 1 < GPW:
            drain_deg(c + 1)
            issue_nodes(c + 1)
    drain_scatter(GPW - 1)


@jax.jit
def _graph_node_features(xt_flat, deg_flat, node_table, degree_table,
                         graph_token):
    mesh = plsc.VectorSubcoreMesh(core_axis_name="c", subcore_axis_name="s")
    out = pl.kernel(
        _sc_body,
        out_type=jax.ShapeDtypeStruct((N_GRAPH, N_NODE + 1, EMB),
                                       jnp.float32),
        mesh=mesh,
        scratch_types=[
            [pltpu.VMEM((IDXC,), jnp.int32) for _ in range(NBUF)],
            [pltpu.VMEM((CHUNK,), jnp.int32) for _ in range(NBUF)],
            [pltpu.VMEM((CHUNK,), jnp.int32) for _ in range(1)],
            pltpu.VMEM((NBUF, CHUNK, EMB), jnp.float32),
            pltpu.VMEM_SHARED((512, EMB), jnp.float32),
            pltpu.VMEM((16, EMB), jnp.float32),
            pltpu.VMEM((16,), jnp.int32),
            [pltpu.SemaphoreType.DMA for _ in range(NBUF)],
            [pltpu.SemaphoreType.DMA for _ in range(NBUF)],
            [pltpu.SemaphoreType.DMA for _ in range(NBUF)],
            [pltpu.SemaphoreType.DMA for _ in range(NBUF)],
        ],
    )(xt_flat, deg_flat, node_table, degree_table, graph_token)
    return out


def kernel(x, degree, node_table, degree_table, graph_token):
    xt_flat = x.astype(jnp.int32).transpose(0, 2, 1).reshape(-1)
    deg_flat = degree.reshape(-1).astype(jnp.int32)
    return _graph_node_features(xt_flat, deg_flat, node_table, degree_table,
                                graph_token)
